# baseline (device time: 69473 ns/iter reference)
import jax
import jax.numpy as jnp
from jax import lax
from jax.experimental import pallas as pl
from jax.experimental.pallas import tpu as pltpu

N_DEV = 4
N_LAYERS = 3
NC = 4


def kernel(x, Win0, Wout0, Win1, Wout1, Win2, Wout2):
    b, d = x.shape
    h = Win0.shape[1]
    C = h // NC

    def body(x_ref, win0, wout0, win1, wout1, win2, wout2, out_ref,
             win_buf, wout_buf, win_sems, wout_sems,
             comm_ref, send_sems, recv_sems):
        my = lax.axis_index("i")
        left = lax.rem(my + N_DEV - 1, N_DEV)
        right = lax.rem(my + 1, N_DEV)
        barrier_sem = pltpu.get_barrier_semaphore()

        wins = [win0, win1, win2]
        wouts = [wout0, wout1, wout2]

        def chunk_dmas(idx, slot):
            layer, c = divmod(idx, NC)
            wdma = pltpu.make_async_copy(
                wins[layer].at[:, pl.ds(c * C, C)],
                win_buf.at[slot],
                win_sems.at[slot],
            )
            odma = pltpu.make_async_copy(
                wouts[layer].at[pl.ds(c * C, C), :],
                wout_buf.at[slot],
                wout_sems.at[slot],
            )
            return wdma, odma

        for dma in chunk_dmas(0, 0):
            dma.start()

        x_cur = x_ref[...]
        for layer in range(N_LAYERS):
            acc = jnp.zeros((b, d), jnp.float32)
            for c in range(NC):
                idx = layer * NC + c
                slot = idx % 2
                if idx + 1 < N_LAYERS * NC:
                    for dma in chunk_dmas(idx + 1, (idx + 1) % 2):
                        dma.start()
                for dma in chunk_dmas(idx, slot):
                    dma.wait()
                hchunk = jnp.maximum(
                    jnp.dot(x_cur, win_buf[slot],
                            preferred_element_type=jnp.float32),
                    0.0,
                )
                acc = acc + jnp.dot(hchunk, wout_buf[slot],
                                    preferred_element_type=jnp.float32)

            for nbr in (left, right):
                pl.semaphore_signal(
                    barrier_sem, inc=1,
                    device_id=(nbr,), device_id_type=pl.DeviceIdType.MESH,
                )
            pl.semaphore_wait(barrier_sem, 2)

            comm_ref[0] = acc
            for hop in range(N_DEV - 1):
                send_slot = hop % 2
                recv_slot = (hop + 1) % 2
                rdma = pltpu.make_async_remote_copy(
                    src_ref=comm_ref.at[send_slot],
                    dst_ref=comm_ref.at[recv_slot],
                    send_sem=send_sems.at[send_slot],
                    recv_sem=recv_sems.at[recv_slot],
                    device_id=(right,),
                    device_id_type=pl.DeviceIdType.MESH,
                )
                rdma.start()
                rdma.wait()
                acc = acc + comm_ref[recv_slot]
            x_cur = acc

        out_ref[...] = x_cur

    weight_spec = pl.BlockSpec(memory_space=pl.ANY)
    return pl.pallas_call(
        body,
        out_shape=jax.ShapeDtypeStruct((b, d), jnp.float32),
        in_specs=[pl.BlockSpec(memory_space=pltpu.VMEM)]
        + [weight_spec] * (2 * N_LAYERS),
        out_specs=pl.BlockSpec(memory_space=pltpu.VMEM),
        scratch_shapes=[
            pltpu.VMEM((2, d, C), jnp.float32),
            pltpu.VMEM((2, C, d), jnp.float32),
            pltpu.SemaphoreType.DMA((2,)),
            pltpu.SemaphoreType.DMA((2,)),
            pltpu.VMEM((2, b, d), jnp.float32),
            pltpu.SemaphoreType.DMA((2,)),
            pltpu.SemaphoreType.DMA((2,)),
        ],
        compiler_params=pltpu.CompilerParams(collective_id=0),
    )(x, Win0, Wout0, Win1, Wout1, Win2, Wout2)


# device time: 31799 ns/iter; 2.1848x vs baseline; 2.1848x over previous
import jax
import jax.numpy as jnp
from jax import lax
from jax.experimental import pallas as pl
from jax.experimental.pallas import tpu as pltpu

N_DEV = 4
N_LAYERS = 3
NC = 4


def kernel(x, Win0, Wout0, Win1, Wout1, Win2, Wout2):
    b, d = x.shape
    h = Win0.shape[1]
    C = h // NC
    CS = d // N_DEV

    def body(x_ref, win0, wout0, win1, wout1, win2, wout2, out_ref,
             win_buf, wout_buf, win_sems, wout_sems, hbuf,
             acc_buf, rs_recv, agsrc_buf, xnext,
             xin_buf, x_sem, out_sem,
             rs_send_sem, rs_recv_sem, ag_send_sem, ag_sems):
        my = lax.axis_index("i")
        nxt = [lax.rem(my + off, N_DEV) for off in (1, 2, 3)]
        order2 = [nxt[1], nxt[0], nxt[2], my]
        order1 = [my, nxt[0], nxt[2], nxt[1]]

        wins = [win0, win1, win2]
        wouts = [wout0, wout1, wout2]

        def win_dma(layer, c):
            return pltpu.make_async_copy(
                wins[layer].at[:, pl.ds(c * C, C)],
                win_buf.at[c],
                win_sems.at[c],
            )

        def wout_dma(layer, k, p):
            return pltpu.make_async_copy(
                wouts[layer].at[:, pl.ds(p * CS, CS)],
                wout_buf.at[k],
                wout_sems.at[k],
            )

        def rs_desc(p):
            return pltpu.make_async_remote_copy(
                src_ref=acc_buf.at[p],
                dst_ref=rs_recv.at[my],
                send_sem=rs_send_sem,
                recv_sem=rs_recv_sem,
                device_id=(p,),
                device_id_type=pl.DeviceIdType.MESH,
            )

        def ag_desc(p):
            return pltpu.make_async_remote_copy(
                src_ref=agsrc_buf,
                dst_ref=xnext.at[my],
                send_sem=ag_send_sem,
                recv_sem=ag_sems.at[my],
                device_id=(p,),
                device_id_type=pl.DeviceIdType.MESH,
            )

        def ag_wait_desc(j):
            return pltpu.make_async_remote_copy(
                src_ref=agsrc_buf,
                dst_ref=xnext.at[j],
                send_sem=ag_send_sem,
                recv_sem=ag_sems.at[j],
                device_id=(j,),
                device_id_type=pl.DeviceIdType.MESH,
            )

        x_dma = pltpu.make_async_copy(x_ref, xin_buf, x_sem)
        x_dma.start()
        for c in range(NC):
            win_dma(0, c).start()
        for k, p in enumerate(order2):
            wout_dma(0, k, p).start()

        barrier_sem = pltpu.get_barrier_semaphore()
        for p in nxt:
            pl.semaphore_signal(
                barrier_sem, inc=1,
                device_id=(p,), device_id_type=pl.DeviceIdType.MESH,
            )
        pl.semaphore_wait(barrier_sem, N_DEV - 1)

        for layer in range(N_LAYERS):
            if layer == 0:
                x_dma.wait()
                xv = xin_buf[...]
                for c in range(NC):
                    win_dma(layer, c).wait()
                    hbuf[:, c * C:(c + 1) * C] = jnp.maximum(
                        jnp.dot(xv, win_buf[c],
                                preferred_element_type=jnp.float32),
                        0.0,
                    )
            else:
                for c in range(NC):
                    win_dma(layer, c).wait()
                for j_idx, j in enumerate(order1):
                    if j_idx > 0:
                        ag_wait_desc(j).wait_recv()
                    xj = xnext[j]
                    for c in range(NC):
                        sl = slice(c * C, (c + 1) * C)
                        contrib = jnp.dot(
                            xj, win_buf[c, pl.ds(j * CS, CS), :],
                            preferred_element_type=jnp.float32)
                        if j_idx == 0:
                            hbuf[:, sl] = contrib
                        elif j_idx < N_DEV - 1:
                            hbuf[:, sl] = hbuf[:, sl] + contrib
                        else:
                            hbuf[:, sl] = jnp.maximum(hbuf[:, sl] + contrib,
                                                      0.0)

            if layer > 0:
                for _ in range(N_DEV - 1):
                    rs_desc(nxt[0]).wait_send()
            hv = hbuf[...]
            for k, p in enumerate(order2):
                wout_dma(layer, k, p).wait()
                part = jnp.dot(hv, wout_buf[k],
                               preferred_element_type=jnp.float32)
                if k < N_DEV - 1:
                    acc_buf[p] = part
                    rs_desc(p).start()
                else:
                    rs_recv[my] = part
                if layer + 1 < N_LAYERS and k < NC:
                    win_dma(layer + 1, k).start()

            for _ in range(N_DEV - 1):
                rs_desc(nxt[0]).wait_recv()
            red = rs_recv[0] + rs_recv[1] + rs_recv[2] + rs_recv[3]

            if layer > 0:
                for _ in range(N_DEV - 1):
                    ag_desc(nxt[0]).wait_send()
            agsrc_buf[...] = red
            xnext[my] = red
            for p in nxt:
                ag_desc(p).start()
            if layer + 1 < N_LAYERS:
                for k, p in enumerate(order2):
                    wout_dma(layer + 1, k, p).start()

        for j in order1[1:]:
            ag_wait_desc(j).wait_recv()
        out_dmas = [
            pltpu.make_async_copy(
                xnext.at[p], out_ref.at[:, pl.ds(p * CS, CS)], out_sem)
            for p in range(N_DEV)
        ]
        for dma in out_dmas:
            dma.start()
        for dma in out_dmas:
            dma.wait()
        for _ in range(N_DEV - 1):
            rs_desc(nxt[0]).wait_send()
        for _ in range(N_DEV - 1):
            ag_desc(nxt[0]).wait_send()

    x, Win0, Wout0, Win1, Wout1, Win2, Wout2 = (
        pltpu.with_memory_space_constraint(w, pltpu.MemorySpace.HBM)
        for w in (x, Win0, Wout0, Win1, Wout1, Win2, Wout2)
    )
    hbm_spec = pl.BlockSpec(memory_space=pltpu.MemorySpace.HBM)
    return pl.pallas_call(
        body,
        out_shape=jax.ShapeDtypeStruct((b, d), jnp.float32),
        in_specs=[hbm_spec] * (1 + 2 * N_LAYERS),
        out_specs=hbm_spec,
        scratch_shapes=[
            pltpu.VMEM((NC, d, C), jnp.float32),
            pltpu.VMEM((N_DEV, h, CS), jnp.float32),
            pltpu.SemaphoreType.DMA((NC,)),
            pltpu.SemaphoreType.DMA((N_DEV,)),
            pltpu.VMEM((b, h), jnp.float32),
            pltpu.VMEM((N_DEV, b, CS), jnp.float32),
            pltpu.VMEM((N_DEV, b, CS), jnp.float32),
            pltpu.VMEM((b, CS), jnp.float32),
            pltpu.VMEM((N_DEV, b, CS), jnp.float32),
            pltpu.VMEM((b, d), jnp.float32),
            pltpu.SemaphoreType.DMA,
            pltpu.SemaphoreType.DMA,
            pltpu.SemaphoreType.DMA,
            pltpu.SemaphoreType.DMA,
            pltpu.SemaphoreType.DMA,
            pltpu.SemaphoreType.DMA((N_DEV,)),
        ],
        compiler_params=pltpu.CompilerParams(collective_id=0),
    )(x, Win0, Wout0, Win1, Wout1, Win2, Wout2)
